# Initial kernel scaffold; baseline (speedup 1.0000x reference)
import jax
import jax.numpy as jnp
from jax.experimental import pallas as pl


def kernel(x, edge_index, W, b):
    raise NotImplementedError("write your pallas kernel here")


# speedup vs baseline: 12.9513x; 12.9513x over previous
import jax
import jax.numpy as jnp
from jax import lax
from jax.experimental import pallas as pl
from jax.experimental.pallas import tpu as pltpu
from jax.experimental.pallas import tpu_sc as plsc

N = 10000
E = 320000
D = 128

NC, NS = 2, 16
NW = NC * NS
CHUNK = 128
NCHUNK = 80
EP = NW * NCHUNK * CHUNK
R = 10240
TRASH = N
RPT = R // NS
BR = 1024

_mesh = plsc.VectorSubcoreMesh(core_axis_name="c", subcore_axis_name="s")


def _deg_body(dst3, degp, idx_v, ones_v, zrow_v, deg_acc):
  c = lax.axis_index("c")
  s = lax.axis_index("s")
  zeros16 = jnp.zeros((16,), jnp.float32)
  ones16 = jnp.ones((16,), jnp.float32)

  def fill_z(i, _):
    zrow_v[pl.ds(i * 16, 16)] = zeros16
    return 0
  lax.fori_loop(0, RPT // 16, fill_z, 0)
  for i in range(CHUNK // 16):
    ones_v[pl.ds(i * 16, 16)] = ones16

  pltpu.sync_copy(zrow_v, deg_acc.at[pl.ds(s * RPT, RPT)])
  plsc.subcore_barrier()

  w = s * NC + c
  pltpu.sync_copy(dst3.at[w], idx_v)

  def scat(j, _):
    pltpu.sync_copy(ones_v, deg_acc.at[idx_v.at[j]], add=True)
    return 0
  lax.fori_loop(0, NCHUNK, scat, 0)

  plsc.subcore_barrier()
  pltpu.sync_copy(deg_acc.at[pl.ds(s * RPT, RPT)],
                  degp.at[c, pl.ds(s * RPT, RPT)])


_sc_deg = pl.kernel(
    _deg_body,
    out_type=jax.ShapeDtypeStruct((NC, R), jnp.float32),
    mesh=_mesh,
    scratch_types=[
        pltpu.VMEM((NCHUNK, CHUNK), jnp.int32),
        pltpu.VMEM((CHUNK,), jnp.float32),
        pltpu.VMEM((RPT,), jnp.float32),
        pltpu.VMEM_SHARED((R,), jnp.float32),
    ],
)


def _scat_body(g, src3, dst3, parts, src_v, dst_v, buf0, buf1, acc,
               sem0, sem1):
  c = lax.axis_index("c")
  s = lax.axis_index("s")
  zeros16 = jnp.zeros((16,), jnp.float32)

  def zrow(i, _):
    for k in range(D // 16):
      buf0[i, pl.ds(k * 16, 16)] = zeros16
    return 0
  lax.fori_loop(0, CHUNK, zrow, 0)
  for j in range(RPT // CHUNK):
    pltpu.sync_copy(buf0, acc.at[pl.ds(s * RPT + j * CHUNK, CHUNK)])

  w = s * NC + c
  pltpu.sync_copy(src3.at[w], src_v)
  pltpu.sync_copy(dst3.at[w], dst_v)
  plsc.subcore_barrier()

  def chunk(j, _):
    pltpu.async_copy(g.at[src_v.at[j]], buf0, sem0).wait()
    pltpu.sync_copy(buf0, acc.at[dst_v.at[j]], add=True)
    return 0
  lax.fori_loop(0, NCHUNK, chunk, 0)

  plsc.subcore_barrier()
  pltpu.sync_copy(acc.at[pl.ds(s * RPT, RPT)],
                  parts.at[c, pl.ds(s * RPT, RPT)])


_sc_scatter = pl.kernel(
    _scat_body,
    out_type=jax.ShapeDtypeStruct((NC, R, D), jnp.float32),
    mesh=_mesh,
    scratch_types=[
        pltpu.VMEM((NCHUNK, CHUNK), jnp.int32),
        pltpu.VMEM((NCHUNK, CHUNK), jnp.int32),
        pltpu.VMEM((CHUNK, D), jnp.float32),
        pltpu.VMEM((CHUNK, D), jnp.float32),
        pltpu.VMEM_SHARED((R, D), jnp.float32),
        pltpu.SemaphoreType.DMA,
        pltpu.SemaphoreType.DMA,
    ],
)


def _mid_body(x_ref, w_ref, b_ref, degp_ref, g_ref, selfb_ref):
  h = jnp.dot(x_ref[...], w_ref[...], preferred_element_type=jnp.float32)
  deg = degp_ref[0, :] + degp_ref[1, :] + 1.0
  dis = lax.rsqrt(deg)
  g_ref[...] = h * dis[:, None]
  selfb_ref[...] = h * (1.0 / deg)[:, None] + b_ref[...]


def _tc_mid(x, W, b2, degp):
  return pl.pallas_call(
      _mid_body,
      grid=(R // BR,),
      in_specs=[
          pl.BlockSpec((BR, D), lambda i: (i, 0)),
          pl.BlockSpec((D, D), lambda i: (0, 0)),
          pl.BlockSpec((1, D), lambda i: (0, 0)),
          pl.BlockSpec((NC, BR), lambda i: (0, i)),
      ],
      out_specs=[
          pl.BlockSpec((BR, D), lambda i: (i, 0)),
          pl.BlockSpec((BR, D), lambda i: (i, 0)),
      ],
      out_shape=[
          jax.ShapeDtypeStruct((N, D), jnp.float32),
          jax.ShapeDtypeStruct((N, D), jnp.float32),
      ],
  )(x, W, b2, degp)


def _final_body(parts_ref, degp_ref, selfb_ref, out_ref):
  deg = degp_ref[0, :] + degp_ref[1, :] + 1.0
  dis = lax.rsqrt(deg)
  psum = parts_ref[0] + parts_ref[1]
  out_ref[...] = selfb_ref[...] + psum * dis[:, None]


def _tc_final(parts, degp, selfb):
  return pl.pallas_call(
      _final_body,
      grid=(R // BR,),
      in_specs=[
          pl.BlockSpec((NC, BR, D), lambda i: (0, i, 0)),
          pl.BlockSpec((NC, BR), lambda i: (0, i)),
          pl.BlockSpec((BR, D), lambda i: (i, 0)),
      ],
      out_specs=pl.BlockSpec((BR, D), lambda i: (i, 0)),
      out_shape=jax.ShapeDtypeStruct((N, D), jnp.float32),
  )(parts, degp, selfb)


@jax.jit
def kernel(x, edge_index, W, b):
  src = edge_index[0].astype(jnp.int32)
  dst = edge_index[1].astype(jnp.int32)
  pad = EP - E
  src3 = jnp.concatenate([src, jnp.zeros((pad,), jnp.int32)]).reshape(
      NW, NCHUNK, CHUNK)
  dst3 = jnp.concatenate([dst, jnp.full((pad,), TRASH, jnp.int32)]).reshape(
      NW, NCHUNK, CHUNK)
  degp = _sc_deg(dst3)
  g, selfb = _tc_mid(x, W, b.reshape(1, D), degp)
  parts = _sc_scatter(g, src3, dst3)
  return _tc_final(parts, degp, selfb)
